# Initial kernel scaffold; baseline (speedup 1.0000x reference)
import jax
import jax.numpy as jnp
from jax.experimental import pallas as pl


def kernel(coords, torsions, fc, periodicity, phase):
    raise NotImplementedError("write your pallas kernel here")


# speedup vs baseline: 8.2216x; 8.2216x over previous
import functools

import jax
import jax.numpy as jnp
from jax import lax
from jax.experimental import pallas as pl
from jax.experimental.pallas import tpu as pltpu
from jax.experimental.pallas import tpu_sc as plsc

_B = 512
_NIDX = (_B * 4) // 128

_CATAN = (
    0.9999999842746153, -0.3333306692040364, 0.19992485341394298,
    -0.14202580379111696, 0.1063678294406024, -0.07495492338820947,
    0.04258803011409887, -0.016005224942998095, 0.002834098973210547,
)
_CSIN = (
    0.9999999999830139, -0.16666666616944775, 0.008333330977974318,
    -0.0001984086158254364, 2.752528772451135e-06, -2.3889498990126157e-08,
)
_PI = 3.14159265358979323846
_TWO_PI = 6.28318530717958647692
_HALF_PI = 1.57079632679489661923


def _rsqrt(x):
    i = lax.bitcast_convert_type(x, jnp.uint32)
    i = jnp.uint32(0x5F3759DF) - (i >> jnp.uint32(1))
    y = lax.bitcast_convert_type(i, jnp.float32)
    half_x = jnp.float32(0.5) * x
    for _ in range(3):
        y = y * (jnp.float32(1.5) - half_x * y * y)
    return y


def _atan2(y, x):
    ax = jnp.abs(x)
    ay = jnp.abs(y)
    hi = jnp.maximum(ax, ay)
    lo = jnp.minimum(ax, ay)
    t = lo / hi
    z = t * t
    p = jnp.float32(_CATAN[-1])
    for c in _CATAN[-2::-1]:
        p = p * z + jnp.float32(c)
    p = p * t
    p = jnp.where(ay > ax, jnp.float32(_HALF_PI) - p, p)
    p = jnp.where(x < jnp.float32(0.0), jnp.float32(_PI) - p, p)
    sign = lax.bitcast_convert_type(y, jnp.uint32) & jnp.uint32(0x80000000)
    mag = lax.bitcast_convert_type(p, jnp.uint32) & jnp.uint32(0x7FFFFFFF)
    return lax.bitcast_convert_type(mag | sign, jnp.float32)


def _cos_wrapped(u):
    u = jnp.where(u < jnp.float32(-_PI), u + jnp.float32(_TWO_PI), u)
    h = jnp.float32(0.5) * u
    w = h * h
    s = jnp.float32(_CSIN[-1])
    for c in _CSIN[-2::-1]:
        s = s * w + jnp.float32(c)
    s = s * h
    return jnp.float32(1.0) - jnp.float32(2.0) * s * s


def _torsion_energy(r, fc_l, per_l, ph_l):
    b1 = [r[1][c] - r[0][c] for c in range(3)]
    b2 = [r[2][c] - r[1][c] for c in range(3)]
    b3 = [r[3][c] - r[2][c] for c in range(3)]
    n1 = [b1[1] * b2[2] - b1[2] * b2[1],
          b1[2] * b2[0] - b1[0] * b2[2],
          b1[0] * b2[1] - b1[1] * b2[0]]
    n2 = [b2[1] * b3[2] - b2[2] * b3[1],
          b2[2] * b3[0] - b2[0] * b3[2],
          b2[0] * b3[1] - b2[1] * b3[0]]
    x = n1[0] * n2[0] + n1[1] * n2[1] + n1[2] * n2[2]
    n1b3 = n1[0] * b3[0] + n1[1] * b3[1] + n1[2] * b3[2]
    b2sq = b2[0] * b2[0] + b2[1] * b2[1] + b2[2] * b2[2]
    y = (b2sq * _rsqrt(b2sq)) * n1b3
    phi = _atan2(y, x)
    return fc_l * (jnp.float32(1.0) + _cos_wrapped(per_l * phi - ph_l))


def _make_sc_kernel(tw, nblk, nw, nc, npad):
    mesh = plsc.VectorSubcoreMesh(core_axis_name="c", subcore_axis_name="s")
    chunk = npad // 16

    @functools.partial(
        pl.kernel,
        out_type=jax.ShapeDtypeStruct((nw, 16), jnp.float32),
        mesh=mesh,
        compiler_params=pltpu.CompilerParams(
            needs_layout_passes=False, use_tc_tiling_on_sc=False),
        scratch_types=[
            pltpu.VMEM((_NIDX, 128), jnp.int32),
            pltpu.VMEM((_B * 4, 8), jnp.float32),
            pltpu.VMEM((_B,), jnp.float32),
            pltpu.VMEM((_B,), jnp.float32),
            pltpu.VMEM((_B,), jnp.float32),
            pltpu.VMEM((16,), jnp.float32),
            pltpu.VMEM_SHARED((npad, 8), jnp.float32),
            pltpu.SemaphoreType.DMA,
        ],
    )
    def torsion_kernel(coords_hbm, tors_hbm, fc_hbm, per_hbm, ph_hbm, out_hbm,
                       idx_v, rows_v, fc_v, per_v, ph_v, acc_v, shared_v, sem):
        cid = lax.axis_index("c")
        sid = lax.axis_index("s")
        wid = sid * nc + cid
        t0w = wid * tw
        acc_v[...] = jnp.zeros((16,), jnp.float32)

        c0 = pl.multiple_of(sid * chunk, 8)
        pltpu.sync_copy(coords_hbm.at[pl.ds(c0, chunk)],
                        shared_v.at[pl.ds(c0, chunk)])
        plsc.subcore_barrier()

        def blk_body(blk, carry):
            t0 = pl.multiple_of(t0w + blk * _B, _B)
            row0 = pl.multiple_of(wid * (tw // 32) + blk * _NIDX, 8)
            pltpu.sync_copy(fc_hbm.at[pl.ds(t0, _B)], fc_v)
            pltpu.sync_copy(per_hbm.at[pl.ds(t0, _B)], per_v)
            pltpu.sync_copy(ph_hbm.at[pl.ds(t0, _B)], ph_v)
            pltpu.sync_copy(tors_hbm.at[pl.ds(row0, _NIDX)], idx_v)
            copies = []
            for j in range(_NIDX):
                copies.append(pltpu.async_copy(
                    shared_v.at[idx_v.at[j]],
                    rows_v.at[pl.ds(j * 128, 128)], sem))
            for cp in copies:
                cp.wait()

            def grp_body(g, carry2):
                base = g * 64
                lane4 = lax.broadcasted_iota(jnp.int32, (16,), 0) * 4
                row0 = base + lane4
                cols = [jnp.full((16,), c, jnp.int32) for c in range(3)]
                r = [[plsc.load_gather(rows_v, [row0 + a, cols[c]])
                      for c in range(3)] for a in range(4)]
                g16 = g * 16
                ene = _torsion_energy(
                    r, fc_v[pl.ds(g16, 16)], per_v[pl.ds(g16, 16)],
                    ph_v[pl.ds(g16, 16)])
                acc_v[...] = acc_v[...] + ene
                return carry2

            lax.fori_loop(0, _B // 16, grp_body, 0)
            return carry

        lax.fori_loop(0, nblk, blk_body, 0)
        pltpu.sync_copy(acc_v, out_hbm.at[wid])

    return torsion_kernel


def kernel(coords, torsions, fc, periodicity, phase):
    n = coords.shape[0]
    m = torsions.shape[0]
    info = plsc.get_sparse_core_info()
    nc, ns = info.num_cores, info.num_subcores
    nw = nc * ns
    tw = -(-(-(-m // nw)) // _B) * _B
    mp = tw * nw
    nblk = tw // _B

    npad = -(-n // 128) * 128
    coords8 = jnp.pad(coords, ((0, npad - n), (0, 5)))
    pad = mp - m
    pad_rows = jnp.broadcast_to(jnp.arange(4, dtype=jnp.int32), (pad, 4))
    tors_p = jnp.concatenate([torsions, pad_rows], axis=0)
    tors2d = tors_p.reshape(mp * 4 // 128, 128)
    zpad = jnp.zeros((pad,), jnp.float32)
    fc_p = jnp.concatenate([fc, zpad])
    per_p = jnp.concatenate([periodicity, zpad])
    ph_p = jnp.concatenate([phase, zpad])

    partials = _make_sc_kernel(tw, nblk, nw, nc, npad)(
        coords8, tors2d, fc_p, per_p, ph_p)
    return jnp.sum(partials)
